# baseline (speedup 1.0000x reference)
import functools

import jax
import jax.numpy as jnp
from jax.experimental import pallas as pl
from jax.experimental.pallas import tpu as pltpu

N = 4096
D = 16
KNB = 15
NSEL = KNB + 1
BR = 128
NPAIR = KNB * (KNB - 1) // 2
HI = jax.lax.Precision.HIGHEST
F32 = jnp.float32
BF16 = jnp.bfloat16
BW = D + 1


def _rowsq(x):
    return jnp.sum(x * x, axis=1, keepdims=True)


def _process_set(e_blk, b_ref, bcat_ref):
    a = jnp.concatenate([e_blk, jnp.ones((BR, 1), F32)], axis=1)
    s = jax.lax.dot_general(a, b_ref[...], (((1,), (1,)), ((), ())),
                            precision=HI, preferred_element_type=F32)
    iota = jax.lax.broadcasted_iota(jnp.int16, s.shape, 1)
    bcat = bcat_ref[...]
    e_t = e_blk.T
    d2Ts = []
    nbTs = []
    for t in range(NSEL):
        idxv = jnp.argmin(s, axis=1)[:, None].astype(jnp.int16)
        mask = iota == idxv
        if t > 0:
            maskf = mask.astype(BF16)
            g2 = jax.lax.dot_general(maskf, bcat, (((1,), (0,)), ((), ())),
                                     preferred_element_type=F32)
            g = g2[:, :BW] + g2[:, BW:]
            nb_t = (g[:, :D] * -0.5).T
            sqj_t = g[:, D:].T
            d2Ts.append(sqj_t - 2.0 * jnp.sum(nb_t * e_t, axis=0,
                                              keepdims=True))
            nbTs.append(nb_t)
        if t < NSEL - 1:
            s = jnp.where(mask, jnp.inf, s)
    sq_t = jnp.sum(e_t * e_t, axis=0, keepdims=True)
    d2 = jnp.maximum(jnp.concatenate(d2Ts, axis=0) + sq_t, 0.0)
    knn_d = jnp.sqrt(d2)
    dmean = jnp.mean(knn_d, axis=0, keepdims=True)
    dn = knn_d / (dmean + 1e-8)
    den = 1.0 / (dmean + 1e-8)
    vhats = []
    for k in range(KNB):
        v = nbTs[k] - e_t
        norm = jnp.sqrt(jnp.sum(v * v, axis=0, keepdims=True))
        vhats.append(v / jnp.maximum(norm, 1e-12))
    return dn, den, vhats


def _body(ec_blk_ref, ec_full_ref, er_blk_ref, er_full_ref,
          s1_ref, sc_ref, sr_ref, scc_ref, srr_ref, scr_ref, s3_ref,
          bc_ref, bccat_ref, br_ref, brcat_ref):
    step = pl.program_id(0)

    @pl.when(step == 0)
    def _build_b():
        for e_ref, b_ref, cat_ref in ((ec_full_ref, bc_ref, bccat_ref),
                                      (er_full_ref, br_ref, brcat_ref)):
            e = e_ref[...]
            b = jnp.concatenate([-2.0 * e, _rowsq(e)], axis=1)
            b_ref[...] = b
            hi = b.astype(BF16)
            lo = (b - hi.astype(F32)).astype(BF16)
            cat_ref[...] = jnp.concatenate([hi, lo], axis=1)

    dnc, denc, vc = _process_set(ec_blk_ref[...], bc_ref, bccat_ref)
    dnr, denr, vr = _process_set(er_blk_ref[...], br_ref, brcat_ref)

    dd = dnc - dnr
    s1 = jnp.sum(dd * dd)
    sc = jnp.sum(denc)
    sr = jnp.sum(denr)
    scc = jnp.sum(denc * denc)
    srr = jnp.sum(denr * denr)
    scr = jnp.sum(denc * denr)

    s3 = jnp.zeros((), F32)
    for k in range(KNB):
        for mth in range(k + 1, KNB):
            csd = (jnp.sum(vc[k] * vc[mth], axis=0, keepdims=True)
                   - jnp.sum(vr[k] * vr[mth], axis=0, keepdims=True))
            s3 = s3 + jnp.sum(csd * csd)

    first = step == 0
    for ref, val in ((s1_ref, s1), (sc_ref, sc), (sr_ref, sr),
                     (scc_ref, scc), (srr_ref, srr), (scr_ref, scr),
                     (s3_ref, s3)):
        prev = jnp.where(first, jnp.zeros((1, 1), F32), ref[...])
        ref[...] = prev + val


@jax.jit
def kernel(embeddings, ref_embeddings):
    assert embeddings.shape == (N, D) and ref_embeddings.shape == (N, D)
    blk = pl.BlockSpec((BR, D), lambda i: (i, 0))
    full = pl.BlockSpec((N, D), lambda i: (0, 0))
    acc = pl.BlockSpec((1, 1), lambda i: (0, 0))
    outs = pl.pallas_call(
        _body,
        grid=(N // BR,),
        in_specs=[blk, full, blk, full],
        out_specs=[acc] * 7,
        out_shape=[jax.ShapeDtypeStruct((1, 1), F32)] * 7,
        scratch_shapes=[pltpu.VMEM((N, BW), F32),
                        pltpu.VMEM((N, 2 * BW), BF16),
                        pltpu.VMEM((N, BW), F32),
                        pltpu.VMEM((N, 2 * BW), BF16)],
    )(embeddings, embeddings, ref_embeddings, ref_embeddings)
    s1, sc, sr, scc, srr, scr, s3 = [o[0, 0] for o in outs]
    n = jnp.float32(N)
    term1 = s1 / (N * KNB)
    mc = sc / n + 1e-8
    mr = sr / n + 1e-8
    term2 = 0.5 * (scc / (mc * mc) - 2.0 * scr / (mc * mr)
                   + srr / (mr * mr)) / n
    term3 = 0.5 * s3 / (n * NPAIR)
    return term1 + term2 + term3


# speedup vs baseline: 1.4907x; 1.4907x over previous
import functools

import jax
import jax.numpy as jnp
from jax.experimental import pallas as pl
from jax.experimental.pallas import tpu as pltpu

N = 4096
D = 16
KNB = 15
NSEL = KNB + 1
BR = 64
NPAIR = KNB * (KNB - 1) // 2
HI = jax.lax.Precision.HIGHEST
F32 = jnp.float32
BF16 = jnp.bfloat16
BW = D + 1


def _rowsq(x):
    return jnp.sum(x * x, axis=1, keepdims=True)


def _process_set(e_blk, b_ref, bcat_ref):
    a = jnp.concatenate([e_blk, jnp.ones((BR, 1), F32)], axis=1)
    s = jax.lax.dot_general(a, b_ref[...], (((1,), (1,)), ((), ())),
                            precision=HI, preferred_element_type=F32)
    iota = jax.lax.broadcasted_iota(jnp.int32, s.shape, 1)
    bcat = bcat_ref[...]
    e_t = e_blk.T
    d2Ts = []
    nbTs = []
    for t in range(NSEL):
        idxv = jnp.argmin(s, axis=1)[:, None]
        mask = iota == idxv
        if t > 0:
            maskf = mask.astype(BF16)
            g2 = jax.lax.dot_general(maskf, bcat, (((1,), (0,)), ((), ())),
                                     preferred_element_type=F32)
            g = g2[:, :BW] + g2[:, BW:]
            nb_t = (g[:, :D] * -0.5).T
            sqj_t = g[:, D:].T
            d2Ts.append(sqj_t - 2.0 * jnp.sum(nb_t * e_t, axis=0,
                                              keepdims=True))
            nbTs.append(nb_t)
        if t < NSEL - 1:
            s = jnp.where(mask, jnp.inf, s)
    sq_t = jnp.sum(e_t * e_t, axis=0, keepdims=True)
    d2 = jnp.maximum(jnp.concatenate(d2Ts, axis=0) + sq_t, 0.0)
    knn_d = jnp.sqrt(d2)
    dmean = jnp.mean(knn_d, axis=0, keepdims=True)
    dn = knn_d / (dmean + 1e-8)
    den = 1.0 / (dmean + 1e-8)
    vhats = []
    for k in range(KNB):
        v = nbTs[k] - e_t
        norm = jnp.sqrt(jnp.sum(v * v, axis=0, keepdims=True))
        vhats.append(v / jnp.maximum(norm, 1e-12))
    return dn, den, vhats


def _body(ec_blk_ref, ec_full_ref, er_blk_ref, er_full_ref,
          s1_ref, sc_ref, sr_ref, scc_ref, srr_ref, scr_ref, s3_ref,
          bc_ref, bccat_ref, br_ref, brcat_ref):
    step = pl.program_id(0)

    @pl.when(step == 0)
    def _build_b():
        for e_ref, b_ref, cat_ref in ((ec_full_ref, bc_ref, bccat_ref),
                                      (er_full_ref, br_ref, brcat_ref)):
            e = e_ref[...]
            b = jnp.concatenate([-2.0 * e, _rowsq(e)], axis=1)
            b_ref[...] = b
            hi = b.astype(BF16)
            lo = (b - hi.astype(F32)).astype(BF16)
            cat_ref[...] = jnp.concatenate([hi, lo], axis=1)

    dnc, denc, vc = _process_set(ec_blk_ref[...], bc_ref, bccat_ref)
    dnr, denr, vr = _process_set(er_blk_ref[...], br_ref, brcat_ref)

    dd = dnc - dnr
    s1 = jnp.sum(dd * dd)
    sc = jnp.sum(denc)
    sr = jnp.sum(denr)
    scc = jnp.sum(denc * denc)
    srr = jnp.sum(denr * denr)
    scr = jnp.sum(denc * denr)

    s3 = jnp.zeros((), F32)
    for k in range(KNB):
        for mth in range(k + 1, KNB):
            csd = (jnp.sum(vc[k] * vc[mth], axis=0, keepdims=True)
                   - jnp.sum(vr[k] * vr[mth], axis=0, keepdims=True))
            s3 = s3 + jnp.sum(csd * csd)

    first = step == 0
    for ref, val in ((s1_ref, s1), (sc_ref, sc), (sr_ref, sr),
                     (scc_ref, scc), (srr_ref, srr), (scr_ref, scr),
                     (s3_ref, s3)):
        prev = jnp.where(first, jnp.zeros((1, 1), F32), ref[...])
        ref[...] = prev + val


@jax.jit
def kernel(embeddings, ref_embeddings):
    assert embeddings.shape == (N, D) and ref_embeddings.shape == (N, D)
    blk = pl.BlockSpec((BR, D), lambda i: (i, 0))
    full = pl.BlockSpec((N, D), lambda i: (0, 0))
    acc = pl.BlockSpec((1, 1), lambda i: (0, 0))
    outs = pl.pallas_call(
        _body,
        grid=(N // BR,),
        in_specs=[blk, full, blk, full],
        out_specs=[acc] * 7,
        out_shape=[jax.ShapeDtypeStruct((1, 1), F32)] * 7,
        scratch_shapes=[pltpu.VMEM((N, BW), F32),
                        pltpu.VMEM((N, 2 * BW), BF16),
                        pltpu.VMEM((N, BW), F32),
                        pltpu.VMEM((N, 2 * BW), BF16)],
    )(embeddings, embeddings, ref_embeddings, ref_embeddings)
    s1, sc, sr, scc, srr, scr, s3 = [o[0, 0] for o in outs]
    n = jnp.float32(N)
    term1 = s1 / (N * KNB)
    mc = sc / n + 1e-8
    mr = sr / n + 1e-8
    term2 = 0.5 * (scc / (mc * mc) - 2.0 * scr / (mc * mr)
                   + srr / (mr * mr)) / n
    term3 = 0.5 * s3 / (n * NPAIR)
    return term1 + term2 + term3


# speedup vs baseline: 2.0049x; 1.3450x over previous
import functools

import jax
import jax.numpy as jnp
from jax.experimental import pallas as pl
from jax.experimental.pallas import tpu as pltpu

N = 4096
D = 16
KNB = 15
NSEL = KNB + 1
BR = 128
NPAIR = KNB * (KNB - 1) // 2
HI = jax.lax.Precision.HIGHEST
F32 = jnp.float32
BF16 = jnp.bfloat16
BW = D + 1


def _rowsq(x):
    return jnp.sum(x * x, axis=1, keepdims=True)


def _process_set(e_blk, b_ref, bcat_ref):
    a = jnp.concatenate([e_blk, jnp.ones((BR, 1), F32)], axis=1)
    s = jax.lax.dot_general(a, b_ref[...], (((1,), (1,)), ((), ())),
                            precision=HI, preferred_element_type=F32)
    iota = jax.lax.broadcasted_iota(jnp.int32, s.shape, 1)
    bcat = bcat_ref[...]
    e_t = e_blk.T
    d2Ts = []
    nbTs = []
    for t in range(NSEL):
        idxv = jnp.argmin(s, axis=1)[:, None]
        mask = iota == idxv
        if t > 0:
            maskf = mask.astype(BF16)
            g2 = jax.lax.dot_general(maskf, bcat, (((1,), (0,)), ((), ())),
                                     preferred_element_type=F32)
            g = g2[:, :BW] + g2[:, BW:]
            nb_t = (g[:, :D] * -0.5).T
            sqj_t = g[:, D:].T
            d2Ts.append(sqj_t - 2.0 * jnp.sum(nb_t * e_t, axis=0,
                                              keepdims=True))
            nbTs.append(nb_t)
        if t < NSEL - 1:
            s = jnp.where(mask, jnp.inf, s)
    sq_t = jnp.sum(e_t * e_t, axis=0, keepdims=True)
    d2 = jnp.maximum(jnp.concatenate(d2Ts, axis=0) + sq_t, 0.0)
    knn_d = jnp.sqrt(d2)
    dmean = jnp.mean(knn_d, axis=0, keepdims=True)
    dn = knn_d / (dmean + 1e-8)
    den = 1.0 / (dmean + 1e-8)
    vhats = []
    for k in range(KNB):
        v = nbTs[k] - e_t
        norm = jnp.sqrt(jnp.sum(v * v, axis=0, keepdims=True))
        vhats.append(v / jnp.maximum(norm, 1e-12))
    return dn, den, vhats


def _body(ec_blk_ref, ec_full_ref, er_blk_ref, er_full_ref,
          s1_ref, sc_ref, sr_ref, scc_ref, srr_ref, scr_ref, s3_ref,
          bc_ref, bccat_ref, br_ref, brcat_ref):
    step = pl.program_id(0)

    @pl.when(step == 0)
    def _build_b():
        for e_ref, b_ref, cat_ref in ((ec_full_ref, bc_ref, bccat_ref),
                                      (er_full_ref, br_ref, brcat_ref)):
            e = e_ref[...]
            b = jnp.concatenate([-2.0 * e, _rowsq(e)], axis=1)
            b_ref[...] = b
            hi = b.astype(BF16)
            lo = (b - hi.astype(F32)).astype(BF16)
            cat_ref[...] = jnp.concatenate([hi, lo], axis=1)

    dnc, denc, vc = _process_set(ec_blk_ref[...], bc_ref, bccat_ref)
    dnr, denr, vr = _process_set(er_blk_ref[...], br_ref, brcat_ref)

    dd = dnc - dnr
    s1 = jnp.sum(dd * dd)
    sc = jnp.sum(denc)
    sr = jnp.sum(denr)
    scc = jnp.sum(denc * denc)
    srr = jnp.sum(denr * denr)
    scr = jnp.sum(denc * denr)

    s3 = jnp.zeros((), F32)
    for k in range(KNB):
        for mth in range(k + 1, KNB):
            csd = (jnp.sum(vc[k] * vc[mth], axis=0, keepdims=True)
                   - jnp.sum(vr[k] * vr[mth], axis=0, keepdims=True))
            s3 = s3 + jnp.sum(csd * csd)

    first = step == 0
    for ref, val in ((s1_ref, s1), (sc_ref, sc), (sr_ref, sr),
                     (scc_ref, scc), (srr_ref, srr), (scr_ref, scr),
                     (s3_ref, s3)):
        prev = jnp.where(first, jnp.zeros((1, 1), F32), ref[...])
        ref[...] = prev + val


@jax.jit
def kernel(embeddings, ref_embeddings):
    assert embeddings.shape == (N, D) and ref_embeddings.shape == (N, D)
    blk = pl.BlockSpec((BR, D), lambda i: (i, 0))
    full = pl.BlockSpec((N, D), lambda i: (0, 0))
    acc = pl.BlockSpec((1, 1), lambda i: (0, 0))
    outs = pl.pallas_call(
        _body,
        grid=(N // BR,),
        in_specs=[blk, full, blk, full],
        out_specs=[acc] * 7,
        out_shape=[jax.ShapeDtypeStruct((1, 1), F32)] * 7,
        scratch_shapes=[pltpu.VMEM((N, BW), F32),
                        pltpu.VMEM((N, 2 * BW), BF16),
                        pltpu.VMEM((N, BW), F32),
                        pltpu.VMEM((N, 2 * BW), BF16)],
    )(embeddings, embeddings, ref_embeddings, ref_embeddings)
    s1, sc, sr, scc, srr, scr, s3 = [o[0, 0] for o in outs]
    n = jnp.float32(N)
    term1 = s1 / (N * KNB)
    mc = sc / n + 1e-8
    mr = sr / n + 1e-8
    term2 = 0.5 * (scc / (mc * mc) - 2.0 * scr / (mc * mr)
                   + srr / (mr * mr)) / n
    term3 = 0.5 * s3 / (n * NPAIR)
    return term1 + term2 + term3
